# Initial kernel scaffold; baseline (speedup 1.0000x reference)
import jax
import jax.numpy as jnp
from jax.experimental import pallas as pl


def kernel(x):
    raise NotImplementedError("write your pallas kernel here")


# speedup vs baseline: 5.7836x; 5.7836x over previous
import numpy as np
import jax
import jax.numpy as jnp
from jax.experimental import pallas as pl
from jax.experimental.pallas import tpu as pltpu

_LEVELS = 256
_EPS = 1e-8
_H = 256
_W = 256
_NPIX = _H * _W
_CHUNK = 8192
_NCHUNKS = _NPIX // _CHUNK
_PAD = 512
_FB = 8


def _offsets8():
    offs = []
    for a in [0.0, 90.0, -45.0, -135.0]:
        for d in [1, 2]:
            offs.append((int(np.round(np.sin(a) * d)),
                         int(np.round(np.cos(a) * d))))
    return offs


_OFFS8 = _offsets8()
_UNIQ = []
_MAP = []
for _o in _OFFS8:
    _dr, _dc = _o
    _cn = (_dr, _dc) if (_dr > 0 or (_dr == 0 and _dc >= 0)) else (-_dr, -_dc)
    if _cn not in _UNIQ:
        _UNIQ.append(_cn)
    _MAP.append(_UNIQ.index(_cn))
_NU = len(_UNIQ)


def _glcm_feat_kernel(x_ref, o_ref):
    img = x_ref[0]
    mn = jnp.min(img)
    mx = jnp.max(img)
    q = jnp.clip(jnp.floor((img - mn) / (mx - mn + _EPS) * 255.0), 0.0, 255.0)
    qb = q.astype(jnp.bfloat16)
    qp = jnp.concatenate([qb, jnp.zeros((1, _PAD), jnp.bfloat16)], axis=1)

    lvl = jax.lax.broadcasted_iota(
        jnp.int32, (_LEVELS, _CHUNK), 0).astype(jnp.bfloat16)
    lane = jax.lax.broadcasted_iota(jnp.int32, (1, _CHUNK), 1)
    one = jnp.bfloat16(1.0)
    zero = jnp.bfloat16(0.0)
    neg = jnp.bfloat16(-1.0)

    hists = [jnp.zeros((_LEVELS, _LEVELS), jnp.float32) for _ in range(_NU)]
    for ci in range(_NCHUNKS):
        base = ci * _CHUNK
        qa = qp[:, base:base + _CHUNK]
        a_oh = jnp.where(lvl == qa, one, zero)
        n = lane + base
        c = jnp.bitwise_and(n, _W - 1)
        r = jnp.right_shift(n, 8)
        for k, (dr, dc) in enumerate(_UNIQ):
            s = dr * _W + dc
            bsh = qp[:, base + s:base + s + _CHUNK]
            ok = (c + dc >= 0) & (c + dc < _W) & (r + dr < _H)
            bm = jnp.where(ok, bsh, neg)
            b_oh = jnp.where(lvl == bm, one, zero)
            hists[k] = hists[k] + jax.lax.dot_general(
                a_oh, b_oh, (((1,), (1,)), ((), ())),
                preferred_element_type=jnp.float32)

    row = jax.lax.broadcasted_iota(
        jnp.int32, (_LEVELS, _LEVELS), 0).astype(jnp.float32)
    col = jax.lax.broadcasted_iota(
        jnp.int32, (_LEVELS, _LEVELS), 1).astype(jnp.float32)
    diff = row - col
    d2 = diff * diff
    hom_w = 1.0 / (1.0 + d2)
    ad = jnp.abs(diff)

    feats = []
    for k in range(_NU):
        h = hists[k]
        g = h + h.T
        ssum = jnp.sum(g)
        P = g / jnp.maximum(ssum, 1.0)
        contrast = jnp.sum(P * d2)
        dissimilarity = jnp.sum(P * ad)
        homogeneity = jnp.sum(P * hom_w)
        asm = jnp.sum(P * P)
        energy = jnp.sqrt(asm)
        mu_i = jnp.sum(row * P)
        mu_j = jnp.sum(col * P)
        di = row - mu_i
        dj = col - mu_j
        var_i = jnp.sum(P * di * di)
        std_i = jnp.sqrt(var_i)
        std_j = jnp.sqrt(jnp.sum(P * dj * dj))
        cov = jnp.sum(P * di * dj)
        denom = std_i * std_j
        correlation = jnp.where(denom < 1e-15, 1.0,
                                cov / jnp.maximum(denom, 1e-15))
        pe = P + _EPS
        entropy = -jnp.sum(pe * jnp.log2(pe))
        feats.append(jnp.stack([contrast, dissimilarity, homogeneity, energy,
                                correlation, asm, entropy, var_i]))
    full = jnp.concatenate([feats[m] for m in _MAP])
    o_ref[...] = full.reshape(1, 1, 64)


def _bcast_kernel(f_ref, o_ref):
    b = pl.program_id(0)
    kk = pl.program_id(1)
    base = kk * _FB
    for i in range(_FB):
        o_ref[0, i] = jnp.full((_H, _W), f_ref[b, base + i], jnp.float32)


def kernel(x):
    b_sz, _, h, w = x.shape
    xf = x.reshape(b_sz, 1, h * w)
    f = pl.pallas_call(
        _glcm_feat_kernel,
        grid=(b_sz,),
        in_specs=[pl.BlockSpec((1, 1, _NPIX), lambda b: (b, 0, 0))],
        out_specs=pl.BlockSpec((1, 1, 64), lambda b: (b, 0, 0)),
        out_shape=jax.ShapeDtypeStruct((b_sz, 1, 64), jnp.float32),
        compiler_params=pltpu.CompilerParams(
            dimension_semantics=("parallel",),
            vmem_limit_bytes=48 * 1024 * 1024,
        ),
    )(xf)
    f2 = f.reshape(b_sz, 64)
    out = pl.pallas_call(
        _bcast_kernel,
        grid=(b_sz, 64 // _FB),
        in_specs=[pl.BlockSpec(memory_space=pltpu.SMEM)],
        out_specs=pl.BlockSpec((1, _FB, h, w), lambda b, k: (b, k, 0, 0)),
        out_shape=jax.ShapeDtypeStruct((b_sz, 64, h, w), jnp.float32),
        compiler_params=pltpu.CompilerParams(
            dimension_semantics=("parallel", "parallel"),
        ),
    )(f2)
    return out
